# Initial kernel scaffold; baseline (speedup 1.0000x reference)
import functools

import jax
import jax.numpy as jnp
from jax import lax
from jax.experimental import pallas as pl
from jax.experimental.pallas import tpu as pltpu
from jax.experimental.pallas import tpu_sc as plsc

_NC, _NS = 2, 16
_NW = _NC * _NS
_CHUNK = 448


def _make_sc_gather(num_rows, vocab, embed):
    rows_per_w = num_rows // _NW
    nchunk = rows_per_w // _CHUNK
    mesh = plsc.VectorSubcoreMesh(
        core_axis_name="c", subcore_axis_name="s",
        num_cores=_NC, num_subcores=_NS)

    @functools.partial(
        pl.kernel,
        out_type=jax.ShapeDtypeStruct((num_rows, embed), jnp.float32),
        mesh=mesh,
        scratch_types=[
            pltpu.VMEM((2, _CHUNK), jnp.int32),
            pltpu.VMEM((2, _CHUNK, embed), jnp.float32),
            pltpu.SemaphoreType.DMA,
            pltpu.SemaphoreType.DMA,
        ],
    )
    def gather(idx_hbm, table_hbm, out_hbm, idx_v, rows_v, sem0, sem1):
        wid = lax.axis_index("s") * _NC + lax.axis_index("c")
        base = wid * rows_per_w
        sems = (sem0, sem1)

        def start(i, b):
            off = base + i * _CHUNK
            pltpu.sync_copy(idx_hbm.at[pl.ds(off, _CHUNK)], idx_v.at[b])
            pltpu.async_copy(table_hbm.at[idx_v.at[b]], rows_v.at[b], sems[b])

        def finish(i, b):
            pltpu.make_async_copy(
                table_hbm.at[idx_v.at[b]], rows_v.at[b], sems[b]).wait()
            off = base + i * _CHUNK
            pltpu.sync_copy(rows_v.at[b], out_hbm.at[pl.ds(off, _CHUNK)])

        start(0, 0)
        start(1, 1)

        def body(g, carry):
            i = 2 * g
            finish(i, 0)
            start(i + 2, 0)
            finish(i + 1, 1)
            start(i + 3, 1)
            return carry

        lax.fori_loop(0, nchunk // 2 - 1, body, 0)
        finish(nchunk - 2, 0)
        finish(nchunk - 1, 1)

    return gather


def _mlp(flat, w1t, b1, w2t, b2):
    batch, feat = flat.shape
    hid = w1t.shape[1]
    out = w2t.shape[1]
    bb = 1024

    def body(f_ref, w1_ref, b1_ref, w2_ref, b2_ref, o_ref):
        h = jnp.dot(f_ref[...], w1_ref[...], preferred_element_type=jnp.float32)
        h = jnp.maximum(h + b1_ref[...], 0.0)
        o_ref[...] = (
            jnp.dot(h, w2_ref[...], preferred_element_type=jnp.float32)
            + b2_ref[...])

    return pl.pallas_call(
        body,
        grid=(batch // bb,),
        in_specs=[
            pl.BlockSpec((bb, feat), lambda i: (i, 0)),
            pl.BlockSpec((feat, hid), lambda i: (0, 0)),
            pl.BlockSpec((1, hid), lambda i: (0, 0)),
            pl.BlockSpec((hid, out), lambda i: (0, 0)),
            pl.BlockSpec((1, out), lambda i: (0, 0)),
        ],
        out_specs=pl.BlockSpec((bb, out), lambda i: (i, 0)),
        out_shape=jax.ShapeDtypeStruct((batch, out), jnp.float32),
    )(flat, w1t, b1.reshape(1, hid), w2t, b2.reshape(1, out))


def kernel(x, table, W1, b1, W2, b2):
    batch, seq = x.shape
    vocab, embed = table.shape
    num_rows = batch * seq
    rows = _make_sc_gather(num_rows, vocab, embed)(x.reshape(num_rows), table)
    flat = rows.reshape(batch, seq * embed)
    return _mlp(flat, W1.T, b1, W2.T, b2)


# speedup vs baseline: 26.9517x; 26.9517x over previous
import functools

import jax
import jax.numpy as jnp
from jax import lax
from jax.experimental import pallas as pl
from jax.experimental.pallas import tpu as pltpu
from jax.experimental.pallas import tpu_sc as plsc

_NC, _NS = 2, 16
_NW = _NC * _NS
_CHUNK = 448


def _make_sc_gather(num_rows, vocab, embed):
    rows_per_w = num_rows // _NW
    nchunk = rows_per_w // _CHUNK
    mesh = plsc.VectorSubcoreMesh(
        core_axis_name="c", subcore_axis_name="s",
        num_cores=_NC, num_subcores=_NS)

    @functools.partial(
        pl.kernel,
        out_type=jax.ShapeDtypeStruct((num_rows, embed), jnp.float32),
        mesh=mesh,
        compiler_params=pltpu.CompilerParams(use_tc_tiling_on_sc=False),
        scratch_types=[
            pltpu.VMEM((_CHUNK,), jnp.int32),
            pltpu.VMEM((_CHUNK,), jnp.int32),
            pltpu.VMEM((_CHUNK, embed), jnp.float32),
            pltpu.VMEM((_CHUNK, embed), jnp.float32),
            pltpu.SemaphoreType.DMA,
            pltpu.SemaphoreType.DMA,
        ],
    )
    def gather(idx_hbm, table_hbm, out_hbm,
               idx_v0, idx_v1, rows_v0, rows_v1, sem0, sem1):
        wid = lax.axis_index("s") * _NC + lax.axis_index("c")
        base = wid * rows_per_w
        idxs = (idx_v0, idx_v1)
        rows = (rows_v0, rows_v1)
        sems = (sem0, sem1)

        def start(i, b):
            off = base + i * _CHUNK
            pltpu.sync_copy(idx_hbm.at[pl.ds(off, _CHUNK)], idxs[b])
            pltpu.async_copy(table_hbm.at[idxs[b]], rows[b], sems[b])

        def finish(i, b):
            pltpu.make_async_copy(
                table_hbm.at[idxs[b]], rows[b], sems[b]).wait()
            off = base + i * _CHUNK
            pltpu.sync_copy(rows[b], out_hbm.at[pl.ds(off, _CHUNK)])

        start(0, 0)
        start(1, 1)

        def body(g, carry):
            i = 2 * g
            finish(i, 0)
            start(i + 2, 0)
            finish(i + 1, 1)
            start(i + 3, 1)
            return carry

        lax.fori_loop(0, nchunk // 2 - 1, body, 0)
        finish(nchunk - 2, 0)
        finish(nchunk - 1, 1)

    return gather


def _mlp(flat, w1t, b1, w2t, b2):
    batch, feat = flat.shape
    hid = w1t.shape[1]
    out = w2t.shape[1]
    bb = 1024

    def body(f_ref, w1_ref, b1_ref, w2_ref, b2_ref, o_ref):
        h = jnp.dot(f_ref[...], w1_ref[...], preferred_element_type=jnp.float32)
        h = jnp.maximum(h + b1_ref[...], 0.0)
        o_ref[...] = (
            jnp.dot(h, w2_ref[...], preferred_element_type=jnp.float32)
            + b2_ref[...])

    return pl.pallas_call(
        body,
        grid=(batch // bb,),
        in_specs=[
            pl.BlockSpec((bb, feat), lambda i: (i, 0)),
            pl.BlockSpec((feat, hid), lambda i: (0, 0)),
            pl.BlockSpec((1, hid), lambda i: (0, 0)),
            pl.BlockSpec((hid, out), lambda i: (0, 0)),
            pl.BlockSpec((1, out), lambda i: (0, 0)),
        ],
        out_specs=pl.BlockSpec((bb, out), lambda i: (i, 0)),
        out_shape=jax.ShapeDtypeStruct((batch, out), jnp.float32),
    )(flat, w1t, b1.reshape(1, hid), w2t, b2.reshape(1, out))


def kernel(x, table, W1, b1, W2, b2):
    batch, seq = x.shape
    vocab, embed = table.shape
    num_rows = batch * seq
    rows = _make_sc_gather(num_rows, vocab, embed)(x.reshape(num_rows), table)
    flat = rows.reshape(batch, seq * embed)
    return _mlp(flat, W1.T, b1, W2.T, b2)
